# baseline (speedup 1.0000x reference)
import functools

import jax
import jax.numpy as jnp
from jax.experimental import pallas as pl

NUM_BEAM = 32
NUM_BIN = 513
NUM_CHANNEL = 16
BATCH = 2048

FT = 27
NCELL = NUM_BIN // FT


def _beam_kernel(x_ref, a_ref, o_ref):
    for t in range(FT):
        rhs = x_ref[:, t].reshape(2 * NUM_CHANNEL, BATCH // 2)
        res = jnp.dot(a_ref[t], rhs,
                      preferred_element_type=jnp.float32)
        o_ref[:, t] = res.reshape(2, NUM_BEAM, BATCH // 2)


@functools.partial(jax.jit, static_argnames=())
def kernel(input, W):
    B, _, F, C = input.shape
    N = W.shape[0]
    wrT = jnp.transpose(W[:, 0], (1, 0, 2))
    wiT = jnp.transpose(W[:, 1], (1, 0, 2))
    top = jnp.stack([wrT, -wiT], axis=2)
    bot = jnp.stack([wiT, wrT], axis=2)
    A = jnp.stack([top, bot], axis=1).reshape(F, 2 * N, 2 * C)

    XT = jnp.transpose(input, (1, 2, 3, 0))
    out = pl.pallas_call(
        _beam_kernel,
        grid=(NCELL, 2),
        in_specs=[
            pl.BlockSpec((2, FT, C, B // 2), lambda i, j: (0, i, 0, j)),
            pl.BlockSpec((FT, 2 * N, 2 * C), lambda i, j: (i, 0, 0)),
        ],
        out_specs=pl.BlockSpec((2, FT, N, B // 2), lambda i, j: (0, i, 0, j)),
        out_shape=jax.ShapeDtypeStruct((2, F, N, B), jnp.float32),
    )(XT, A)
    return jnp.transpose(out, (3, 0, 1, 2))


# speedup vs baseline: 1.0279x; 1.0279x over previous
import functools

import jax
import jax.numpy as jnp
from jax.experimental import pallas as pl

NUM_BEAM = 32
NUM_BIN = 513
NUM_CHANNEL = 16
BATCH = 2048

FT = 27
NCELL = NUM_BIN // FT


def _beam_kernel(x_ref, a_ref, o_ref):
    for t in range(FT):
        rhs = x_ref[:, t].reshape(2 * NUM_CHANNEL, BATCH)
        res = jnp.dot(a_ref[t], rhs,
                      preferred_element_type=jnp.float32)
        o_ref[:, t] = res.reshape(2, NUM_BEAM, BATCH)


@functools.partial(jax.jit, static_argnames=())
def kernel(input, W):
    B, _, F, C = input.shape
    N = W.shape[0]
    wrT = jnp.transpose(W[:, 0], (1, 0, 2))
    wiT = jnp.transpose(W[:, 1], (1, 0, 2))
    top = jnp.stack([wrT, -wiT], axis=2)
    bot = jnp.stack([wiT, wrT], axis=2)
    A = jnp.stack([top, bot], axis=1).reshape(F, 2 * N, 2 * C)

    XT = jnp.transpose(input, (1, 2, 3, 0))
    out = pl.pallas_call(
        _beam_kernel,
        grid=(NCELL,),
        in_specs=[
            pl.BlockSpec((2, FT, C, B), lambda i: (0, i, 0, 0)),
            pl.BlockSpec((FT, 2 * N, 2 * C), lambda i: (i, 0, 0)),
        ],
        out_specs=pl.BlockSpec((2, FT, N, B), lambda i: (0, i, 0, 0)),
        out_shape=jax.ShapeDtypeStruct((2, F, N, B), jnp.float32),
    )(XT, A)
    return jnp.transpose(out, (3, 0, 1, 2))
